# Initial kernel scaffold; baseline (speedup 1.0000x reference)
import jax
import jax.numpy as jnp
from jax.experimental import pallas as pl


def kernel(id, sector, stage, region, deal_size, revenue_multiple, growth_rate, profitability, team_experience, market_size, deal_table, sector_table, stage_table, region_table, W1, b1, g1, beta1, W2, b2, g2, beta2):
    raise NotImplementedError("write your pallas kernel here")


# speedup vs baseline: 1.0861x; 1.0861x over previous
import jax
import jax.numpy as jnp
from jax import lax
from jax.experimental import pallas as pl
from jax.experimental.pallas import tpu as pltpu
from jax.experimental.pallas import tpu_sc as plsc

B = 4096
EMB = 64
H1, H2 = 256, 128
NC, NS = 2, 16
NW = NC * NS
BPW = B // NW


def _sc_gather_body(idx_hbm, table_hbm, out_hbm, idx_v, rows_v, sem):
    wid = lax.axis_index("s") * NC + lax.axis_index("c")
    base = wid * BPW
    pltpu.sync_copy(idx_hbm.at[pl.ds(base, BPW)], idx_v)
    pltpu.async_copy(table_hbm.at[idx_v], rows_v, sem).wait()
    pltpu.sync_copy(rows_v, out_hbm.at[pl.ds(base, BPW)])


def _sc_gather(idx, table):
    mesh = plsc.VectorSubcoreMesh(core_axis_name="c", subcore_axis_name="s")
    return pl.kernel(
        _sc_gather_body,
        mesh=mesh,
        out_type=jax.ShapeDtypeStruct((B, EMB), jnp.float32),
        compiler_params=pltpu.CompilerParams(use_tc_tiling_on_sc=False),
        scratch_types=[
            pltpu.VMEM((BPW,), jnp.int32),
            pltpu.VMEM((BPW, EMB), jnp.float32),
            pltpu.SemaphoreType.DMA,
        ],
    )(idx, table)


def _mlp_body(id_emb, sec, stg, reg, nums,
              sec_t, stg_t, reg_t,
              w1a, w1s, w1t, w1r, w1n, b1, g1, beta1,
              w2, b2, g2, beta2, out):
    f32 = jnp.float32

    sec_oh = (sec[...] == lax.broadcasted_iota(jnp.int32, (B, 32), 1)).astype(f32)
    stg_oh = (stg[...] == lax.broadcasted_iota(jnp.int32, (B, 16), 1)).astype(f32)
    reg_oh = (reg[...] == lax.broadcasted_iota(jnp.int32, (B, 24), 1)).astype(f32)

    sec_w = jnp.dot(sec_t[...], w1s[...], preferred_element_type=f32)
    stg_w = jnp.dot(stg_t[...], w1t[...], preferred_element_type=f32)
    reg_w = jnp.dot(reg_t[...], w1r[...], preferred_element_type=f32)

    h = (jnp.dot(id_emb[...], w1a[...], preferred_element_type=f32)
         + jnp.dot(sec_oh, sec_w, preferred_element_type=f32)
         + jnp.dot(stg_oh, stg_w, preferred_element_type=f32)
         + jnp.dot(reg_oh, reg_w, preferred_element_type=f32)
         + jnp.dot(nums[...], w1n[...], preferred_element_type=f32)
         + b1[...])
    h = jnp.maximum(h, 0.0)
    mu = jnp.mean(h, axis=0, keepdims=True)
    var = jnp.mean((h - mu) ** 2, axis=0, keepdims=True)
    h = g1[...] * (h - mu) * lax.rsqrt(var + 1e-5) + beta1[...]

    h = jnp.dot(h, w2[...], preferred_element_type=f32) + b2[...]
    h = jnp.maximum(h, 0.0)
    mu2 = jnp.mean(h, axis=0, keepdims=True)
    var2 = jnp.mean((h - mu2) ** 2, axis=0, keepdims=True)
    h = g2[...] * (h - mu2) * lax.rsqrt(var2 + 1e-5) + beta2[...]

    norm = jnp.sqrt(jnp.sum(h * h, axis=1, keepdims=True))
    out[...] = h / jnp.maximum(norm, 1e-12)


def _mlp(id_emb, sector, stage, region, nums, sector_table, stage_table,
         region_table, W1, b1, g1, beta1, W2, b2, g2, beta2):
    f32 = jnp.float32
    args = (
        id_emb,
        sector.reshape(B, 1).astype(jnp.int32),
        stage.reshape(B, 1).astype(jnp.int32),
        region.reshape(B, 1).astype(jnp.int32),
        nums,
        sector_table, stage_table, region_table,
        W1[0:64], W1[64:80], W1[80:96], W1[96:112], W1[112:118],
        b1.reshape(1, H1), g1.reshape(1, H1), beta1.reshape(1, H1),
        W2,
        b2.reshape(1, H2), g2.reshape(1, H2), beta2.reshape(1, H2),
    )
    return pl.pallas_call(
        _mlp_body,
        out_shape=jax.ShapeDtypeStruct((B, H2), f32),
    )(*args)


def kernel(id, sector, stage, region, deal_size, revenue_multiple,
           growth_rate, profitability, team_experience, market_size,
           deal_table, sector_table, stage_table, region_table,
           W1, b1, g1, beta1, W2, b2, g2, beta2):
    id_emb = _sc_gather(id.astype(jnp.int32), deal_table)
    nums = jnp.stack([deal_size, revenue_multiple, growth_rate, profitability,
                      team_experience, market_size], axis=-1)
    return _mlp(id_emb, sector, stage, region, nums, sector_table,
                stage_table, region_table, W1, b1, g1, beta1, W2, b2, g2, beta2)
